# Initial kernel scaffold; baseline (speedup 1.0000x reference)
import jax
import jax.numpy as jnp
from jax.experimental import pallas as pl


def kernel(x, embedding):
    raise NotImplementedError("write your pallas kernel here")


# speedup vs baseline: 3.1818x; 3.1818x over previous
import functools

import jax
import jax.numpy as jnp
from jax import lax
from jax.experimental import pallas as pl
from jax.experimental.pallas import tpu as pltpu
from jax.experimental.pallas import tpu_sc as plsc

_DIM = 64
_CHUNK = 128


def _gather_body(x_hbm, emb_hbm, out_hbm, idx_v, rows_v, sem, *, per_w, n_chunks):
    wid = lax.axis_index("s") * 2 + lax.axis_index("c")
    base = wid * per_w

    def body(c, carry):
        row0 = base + c * _CHUNK
        pltpu.sync_copy(x_hbm.at[pl.ds(row0, _CHUNK)], idx_v)
        pltpu.async_copy(emb_hbm.at[idx_v], rows_v, sem).wait()
        pltpu.sync_copy(rows_v, out_hbm.at[pl.ds(row0, _CHUNK)])
        return carry

    lax.fori_loop(0, n_chunks, body, 0)


@functools.partial(jax.jit, static_argnames=("n",))
def _sc_gather(x_flat, embedding, n):
    nw = 32
    per_w = n // nw
    n_chunks = per_w // _CHUNK
    mesh = plsc.VectorSubcoreMesh(core_axis_name="c", subcore_axis_name="s")
    kfn = pl.kernel(
        functools.partial(_gather_body, per_w=per_w, n_chunks=n_chunks),
        mesh=mesh,
        out_type=jax.ShapeDtypeStruct((n, _DIM), jnp.float32),
        scratch_types=[
            pltpu.VMEM((_CHUNK,), jnp.int32),
            pltpu.VMEM((_CHUNK, _DIM), jnp.float32),
            pltpu.SemaphoreType.DMA,
        ],
        compiler_params=pltpu.CompilerParams(use_tc_tiling_on_sc=False),
    )
    return kfn(x_flat, embedding)


def kernel(x, embedding):
    b, l = x.shape
    n = b * l
    out = _sc_gather(x.reshape(n).astype(jnp.int32), embedding, n)
    return out.reshape(b, l, _DIM)


# speedup vs baseline: 4.2501x; 1.3357x over previous
import functools

import jax
import jax.numpy as jnp
from jax import lax
from jax.experimental import pallas as pl
from jax.experimental.pallas import tpu as pltpu
from jax.experimental.pallas import tpu_sc as plsc

_DIM = 64
_SUB = 128
_K = 4
_NBUF = 3
_GROUP = _SUB * _K


def _gather_body(x_hbm, emb_hbm, out_hbm, idx_v, rows_v,
                 sg0, sg1, sg2, so0, so1, so2, *, n_chunks, n_groups):
    wid = lax.axis_index("s") * 2 + lax.axis_index("c")
    chunk0 = wid * n_chunks
    sem_g = (sg0, sg1, sg2)
    sem_o = (so0, so1, so2)

    pltpu.sync_copy(x_hbm.at[pl.ds(chunk0 * 1, n_chunks)], idx_v)

    def fire_gathers(g, b):
        for j in range(_K):
            pltpu.async_copy(
                emb_hbm.at[idx_v.at[g * _K + j]], rows_v.at[b, j], sem_g[b])

    def drain_gathers(b):
        for j in range(_K):
            pltpu.make_async_copy(
                emb_hbm.at[idx_v.at[0]], rows_v.at[b, j], sem_g[b]).wait()

    def fire_write(g, b):
        pltpu.async_copy(
            rows_v.at[b], out_hbm.at[pl.ds(chunk0 + g * _K, _K)], sem_o[b])

    def wait_write(b):
        pltpu.make_async_copy(
            rows_v.at[b], out_hbm.at[pl.ds(0, _K)], sem_o[b]).wait()

    fire_gathers(0, 0)
    fire_gathers(1, 1)
    drain_gathers(0)
    fire_write(0, 0)
    fire_gathers(2, 2)
    drain_gathers(1)
    fire_write(1, 1)

    n_steady = ((n_groups - 3) // 3) * 3

    def body(i, carry):
        for r in range(3):
            g = 3 + 3 * i + r
            b = r
            wait_write(b)
            fire_gathers(g, b)
            drain_gathers((r + 2) % 3)
            fire_write(g - 1, (r + 2) % 3)
        return carry

    lax.fori_loop(0, n_steady // 3, body, 0)

    for g in range(3 + n_steady, n_groups):
        b = g % 3
        wait_write(b)
        fire_gathers(g, b)
        drain_gathers((b + 2) % 3)
        fire_write(g - 1, (b + 2) % 3)
    b_last = (n_groups - 1) % 3
    drain_gathers(b_last)
    fire_write(n_groups - 1, b_last)
    for db in range(3):
        wait_write((b_last + 1 + db) % 3)


@functools.partial(jax.jit, static_argnames=("n",))
def _sc_gather(x2d, embedding, n):
    nw = 32
    per_w = n // nw
    n_chunks = per_w // _SUB
    n_groups = per_w // _GROUP
    mesh = plsc.VectorSubcoreMesh(core_axis_name="c", subcore_axis_name="s")
    kfn = pl.kernel(
        functools.partial(_gather_body, n_chunks=n_chunks, n_groups=n_groups),
        mesh=mesh,
        out_type=jax.ShapeDtypeStruct((n // _SUB, _SUB, _DIM), jnp.float32),
        scratch_types=[
            pltpu.VMEM((n_chunks, _SUB), jnp.int32),
            pltpu.VMEM((_NBUF, _K, _SUB, _DIM), jnp.float32),
        ] + [pltpu.SemaphoreType.DMA] * 6,
        compiler_params=pltpu.CompilerParams(use_tc_tiling_on_sc=False),
    )
    return kfn(x2d, embedding)


def kernel(x, embedding):
    b, l = x.shape
    n = b * l
    x2d = x.reshape(n // _SUB, _SUB).astype(jnp.int32)
    out = _sc_gather(x2d, embedding, n)
    return out.reshape(b, l, _DIM)


# speedup vs baseline: 4.2592x; 1.0021x over previous
import functools

import jax
import jax.numpy as jnp
from jax import lax
from jax.experimental import pallas as pl
from jax.experimental.pallas import tpu as pltpu
from jax.experimental.pallas import tpu_sc as plsc

_DIM = 64
_SUB = 512
_K = 1
_NBUF = 3
_GROUP = _SUB * _K


def _gather_body(x_hbm, emb_hbm, out_hbm, idx_v, rows_v,
                 sg0, sg1, sg2, so0, so1, so2, *, n_chunks, n_groups):
    wid = lax.axis_index("s") * 2 + lax.axis_index("c")
    chunk0 = wid * n_chunks
    sem_g = (sg0, sg1, sg2)
    sem_o = (so0, so1, so2)

    pltpu.sync_copy(x_hbm.at[pl.ds(chunk0 * 1, n_chunks)], idx_v)

    def fire_gathers(g, b):
        for j in range(_K):
            pltpu.async_copy(
                emb_hbm.at[idx_v.at[g * _K + j]], rows_v.at[b, j], sem_g[b])

    def drain_gathers(b):
        for j in range(_K):
            pltpu.make_async_copy(
                emb_hbm.at[idx_v.at[0]], rows_v.at[b, j], sem_g[b]).wait()

    def fire_write(g, b):
        pltpu.async_copy(
            rows_v.at[b], out_hbm.at[pl.ds(chunk0 + g * _K, _K)], sem_o[b])

    def wait_write(b):
        pltpu.make_async_copy(
            rows_v.at[b], out_hbm.at[pl.ds(0, _K)], sem_o[b]).wait()

    fire_gathers(0, 0)
    fire_gathers(1, 1)
    drain_gathers(0)
    fire_write(0, 0)
    fire_gathers(2, 2)
    drain_gathers(1)
    fire_write(1, 1)

    n_steady = ((n_groups - 3) // 3) * 3

    def body(i, carry):
        for r in range(3):
            g = 3 + 3 * i + r
            b = r
            wait_write(b)
            fire_gathers(g, b)
            drain_gathers((r + 2) % 3)
            fire_write(g - 1, (r + 2) % 3)
        return carry

    lax.fori_loop(0, n_steady // 3, body, 0)

    for g in range(3 + n_steady, n_groups):
        b = g % 3
        wait_write(b)
        fire_gathers(g, b)
        drain_gathers((b + 2) % 3)
        fire_write(g - 1, (b + 2) % 3)
    b_last = (n_groups - 1) % 3
    drain_gathers(b_last)
    fire_write(n_groups - 1, b_last)
    for db in range(3):
        wait_write((b_last + 1 + db) % 3)


@functools.partial(jax.jit, static_argnames=("n",))
def _sc_gather(x2d, embedding, n):
    nw = 32
    per_w = n // nw
    n_chunks = per_w // _SUB
    n_groups = per_w // _GROUP
    mesh = plsc.VectorSubcoreMesh(core_axis_name="c", subcore_axis_name="s")
    kfn = pl.kernel(
        functools.partial(_gather_body, n_chunks=n_chunks, n_groups=n_groups),
        mesh=mesh,
        out_type=jax.ShapeDtypeStruct((n // _SUB, _SUB, _DIM), jnp.float32),
        scratch_types=[
            pltpu.VMEM((n_chunks, _SUB), jnp.int32),
            pltpu.VMEM((_NBUF, _K, _SUB, _DIM), jnp.float32),
        ] + [pltpu.SemaphoreType.DMA] * 6,
        compiler_params=pltpu.CompilerParams(use_tc_tiling_on_sc=False),
    )
    return kfn(x2d, embedding)


def kernel(x, embedding):
    b, l = x.shape
    n = b * l
    x2d = x.reshape(n // _SUB, _SUB).astype(jnp.int32)
    out = _sc_gather(x2d, embedding, n)
    return out.reshape(b, l, _DIM)


# speedup vs baseline: 5.5744x; 1.3088x over previous
import functools

import jax
import jax.numpy as jnp
from jax import lax
from jax.experimental import pallas as pl
from jax.experimental.pallas import tpu as pltpu
from jax.experimental.pallas import tpu_sc as plsc

_DIM = 64
_PAD = 128
_NBUF = 3


def _gather_body(x_hbm, emb_hbm, out_hbm, idx_v, rows_v,
                 sg0, sg1, sg2, so0, so1, so2, *, seq, n_batches):
    wid = lax.axis_index("s") * 2 + lax.axis_index("c")
    b0 = wid * n_batches
    sem_g = (sg0, sg1, sg2)
    sem_o = (so0, so1, so2)

    pltpu.sync_copy(x_hbm.at[pl.ds(b0 * seq, n_batches * seq)], idx_v)

    def fire_gather(g, b):
        pltpu.async_copy(
            emb_hbm.at[idx_v.at[pl.ds(g * seq, seq)]], rows_v.at[b], sem_g[b])

    def drain_gather(b):
        pltpu.make_async_copy(
            emb_hbm.at[idx_v.at[pl.ds(0, seq)]], rows_v.at[b], sem_g[b]).wait()

    def fire_write(g, b):
        pltpu.async_copy(rows_v.at[b], out_hbm.at[b0 + g], sem_o[b])

    def wait_write(b):
        pltpu.make_async_copy(rows_v.at[b], out_hbm.at[0], sem_o[b]).wait()

    fire_gather(0, 0)
    fire_gather(1, 1)
    drain_gather(0)
    fire_write(0, 0)
    fire_gather(2, 2)
    drain_gather(1)
    fire_write(1, 1)

    n_groups = n_batches
    n_steady = ((n_groups - 3) // 3) * 3

    def body(i, carry):
        for r in range(3):
            g = 3 + 3 * i + r
            b = r
            wait_write(b)
            fire_gather(g, b)
            drain_gather((r + 2) % 3)
            fire_write(g - 1, (r + 2) % 3)
        return carry

    lax.fori_loop(0, n_steady // 3, body, 0)

    for g in range(3 + n_steady, n_groups):
        b = g % 3
        wait_write(b)
        fire_gather(g, b)
        drain_gather((b + 2) % 3)
        fire_write(g - 1, (b + 2) % 3)
    b_last = (n_groups - 1) % 3
    drain_gather(b_last)
    fire_write(n_groups - 1, b_last)
    for db in range(3):
        wait_write((b_last + 1 + db) % 3)


@functools.partial(jax.jit, static_argnames=("bsz", "seq"))
def _sc_gather(x_flat, emb_pad, bsz, seq):
    nw = 32
    n_batches = bsz // nw
    mesh = plsc.VectorSubcoreMesh(core_axis_name="c", subcore_axis_name="s")
    kfn = pl.kernel(
        functools.partial(_gather_body, seq=seq, n_batches=n_batches),
        mesh=mesh,
        out_type=jax.ShapeDtypeStruct((bsz, seq, _PAD), jnp.float32),
        scratch_types=[
            pltpu.VMEM((n_batches * seq,), jnp.int32),
            pltpu.VMEM((_NBUF, seq, _PAD), jnp.float32),
        ] + [pltpu.SemaphoreType.DMA] * 6,
        compiler_params=pltpu.CompilerParams(use_tc_tiling_on_sc=True),
    )
    return kfn(x_flat, emb_pad)


def kernel(x, embedding):
    bsz, seq = x.shape
    x_flat = x.reshape(bsz * seq).astype(jnp.int32)
    emb_pad = jnp.pad(embedding, ((0, 0), (0, _PAD - _DIM)))
    out_p = _sc_gather(x_flat, emb_pad, bsz, seq)
    return out_p[:, :, :_DIM]
